# baseline (speedup 1.0000x reference)
import functools

import jax
import jax.numpy as jnp
from jax import lax
from jax.experimental import pallas as pl
from jax.experimental.pallas import tpu as pltpu
from jax.experimental.pallas import tpu_sc as plsc


def _mm_body(x_ref, w_ref, o_ref):
    o_ref[...] = jnp.dot(x_ref[...], w_ref[...],
                         preferred_element_type=jnp.float32)


def _matmul(x, W):
    n, _ = x.shape
    d_out = W.shape[1]
    return pl.pallas_call(
        _mm_body,
        out_shape=jax.ShapeDtypeStruct((n, d_out), jnp.float32),
    )(x, W)


def _comb_body(p_ref, b_ref, o_ref):
    o_ref[...] = p_ref[0] + p_ref[1] + b_ref[...]


def _combine(partial, b):
    _, n, d_out = partial.shape
    return pl.pallas_call(
        _comb_body,
        out_shape=jax.ShapeDtypeStruct((n, d_out), jnp.float32),
    )(partial, b.reshape(1, d_out))


def _sc_scatter(feats, srcp, dstp, etcp, alpha_pad):
    n, d = feats.shape
    e_pad = srcp.shape[0]
    info = plsc.get_sparse_core_info()
    nc, ns = info.num_cores, info.num_subcores
    nw = nc * ns
    ept = e_pad // nw
    B = 128
    nchunk = ept // B
    nseg = 2
    epth = ept // nseg
    U = 80
    nu = n // U
    upt = -(-nu // ns)
    n_acc = n + U
    ng = d // 16

    mesh = plsc.VectorSubcoreMesh(core_axis_name="c", subcore_axis_name="s")

    @functools.partial(
        pl.kernel,
        mesh=mesh,
        compiler_params=pltpu.CompilerParams(needs_layout_passes=False),
        out_type=jax.ShapeDtypeStruct((nc, n, d), jnp.float32),
        scratch_types=[
            pltpu.VMEM((B,), jnp.int32),
            pltpu.VMEM((B,), jnp.int32),
            pltpu.VMEM((B,), jnp.int32),
            pltpu.VMEM((B,), jnp.int32),
            pltpu.VMEM((32,), jnp.float32),
            pltpu.VMEM((B, d), jnp.float32),
            pltpu.VMEM((B, d), jnp.float32),
            pltpu.VMEM_SHARED((n_acc, d), jnp.float32),
            pltpu.SemaphoreType.DMA,
            pltpu.SemaphoreType.DMA,
            pltpu.SemaphoreType.DMA,
            pltpu.SemaphoreType.DMA,
        ],
    )
    def k(feats_hbm, src_hbm, dst_hbm, etc_hbm, alpha_hbm, out_hbm,
          srcv, dsti0, dsti1, etcv, alphav, rows0, rows1, accum,
          g0, g1, d0, d1):
        cid = lax.axis_index("c")
        sid = lax.axis_index("s")
        wid = sid * nc + cid

        base = wid * ept
        pltpu.sync_copy(alpha_hbm, alphav)

        zero16 = jnp.zeros((16,), jnp.float32)
        for r in range(U):
            for g in range(ng):
                rows0[r, pl.ds(g * 16, 16)] = zero16
        for j in range(upt):
            u = sid + j * ns
            @pl.when(u < nu)
            def _():
                pltpu.sync_copy(rows0.at[pl.ds(0, U)],
                                accum.at[pl.ds(pl.multiple_of(u * U, 16), U)])
        plsc.subcore_barrier()

        at0 = alphav[pl.ds(0, 16)]
        at1 = alphav[pl.ds(16, 16)]

        def _splat(vec, lane):
            idx = jnp.full((16,), lane, jnp.int32)
            return vec.at[idx].get(mode="promise_in_bounds")

        def _lookup(v):
            lo = at0.at[jnp.minimum(v, 15)].get(mode="promise_in_bounds")
            hi = at1.at[jnp.maximum(v - 16, 0)].get(mode="promise_in_bounds")
            return jnp.where(v < 16, lo, hi)

        def _gather_issue(c, buf, sem):
            pltpu.async_copy(feats_hbm.at[srcv.at[pl.ds(c * B, B)]], buf, sem)

        def _gather_wait(c, buf, sem):
            pltpu.make_async_copy(feats_hbm.at[srcv.at[pl.ds(c * B, B)]],
                                  buf, sem).wait()

        def _dst_issue(gc, dbuf, sem):
            pltpu.async_copy(dst_hbm.at[pl.ds(base + gc * B, B)], dbuf, sem)

        def _dst_wait(gc, dbuf, sem):
            pltpu.make_async_copy(dst_hbm.at[pl.ds(base + gc * B, B)],
                                  dbuf, sem).wait()

        def _scatter(buf, dbuf):
            pltpu.sync_copy(buf, accum.at[dbuf], add=True)

        def _scale(buf):
            for j in range(B // 16):
                v = etcv[pl.ds(j * 16, 16)]
                av = _lookup(v & 255) + _lookup(v >> 8)
                for l in range(16):
                    r = j * 16 + l
                    s = _splat(av, l)
                    for g in range(ng):
                        buf[r, pl.ds(g * 16, 16)] = (
                            buf[r, pl.ds(g * 16, 16)] * s)

        def step(c, carry):
            goff = base + c * B
            pltpu.sync_copy(src_hbm.at[pl.ds(goff, B)], srcv)
            pltpu.sync_copy(dst_hbm.at[pl.ds(goff, B)], dsti0)
            pltpu.sync_copy(etc_hbm.at[pl.ds(goff, B)], etcv)
            pltpu.async_copy(feats_hbm.at[srcv], rows0, g0).wait()
            _scale(rows0)
            _scatter(rows0, dsti0)
            return carry

        lax.fori_loop(0, nchunk, step, 0)

        plsc.subcore_barrier()
        for j in range(upt):
            u = sid + j * ns
            @pl.when(u < nu)
            def _():
                r0 = pl.multiple_of(u * U, 16)
                pltpu.sync_copy(accum.at[pl.ds(r0, U)],
                                out_hbm.at[cid, pl.ds(r0, U)])

    return k(feats, srcp, dstp, etcp, alpha_pad)


def kernel(x, edge_index, all_edge_type, W, alpha_table, b):
    n = x.shape[0]
    e = all_edge_type.shape[0]
    t = (e - n) // 2
    src = edge_index[0]
    dst = edge_index[1]
    ett = jnp.concatenate([all_edge_type[t:2 * t],
                           all_edge_type[:t],
                           all_edge_type[2 * t:]])
    alpha_pad = jnp.pad(alpha_table[:, 0], (0, 32 - alpha_table.shape[0]))
    B, nw = 128, 32
    e_pad = -(-e // (B * nw)) * B * nw
    pad = e_pad - e
    zi = jnp.zeros((pad,), jnp.int32)
    srcp = jnp.concatenate([src, zi])
    dstp = jnp.concatenate([dst, jnp.full((pad,), n, jnp.int32)])
    etc = all_edge_type | (ett << 8)
    etcp = jnp.concatenate([etc, zi])
    feats = _matmul(x, W)
    partial = _sc_scatter(feats, srcp, dstp, etcp, alpha_pad)
    return _combine(partial, b)
